# baseline (speedup 1.0000x reference)
import functools

import jax
import jax.numpy as jnp
from jax import lax
from jax.experimental import pallas as pl
from jax.experimental.pallas import tpu as pltpu
from jax.experimental.pallas import tpu_sc as plsc

D = 32
L = 16
IDX_CHUNK = 128


@functools.lru_cache(maxsize=None)
def _build(B):
    info = plsc.get_sparse_core_info()
    NC, NS = info.num_cores, info.num_subcores
    NW = NC * NS
    BW = B // NW
    NCH = BW // IDX_CHUNK
    mesh = plsc.VectorSubcoreMesh(core_axis_name="c", subcore_axis_name="s")

    @functools.partial(
        pl.kernel,
        mesh=mesh,
        compiler_params=pltpu.CompilerParams(
            needs_layout_passes=False, use_tc_tiling_on_sc=False),
        out_type=jax.ShapeDtypeStruct((B,), jnp.float32),
        scratch_types=[
            pltpu.VMEM((NCH, IDX_CHUNK), jnp.int32),
            pltpu.VMEM((NCH, IDX_CHUNK), jnp.int32),
            pltpu.VMEM((NCH, IDX_CHUNK), jnp.int32),
            pltpu.VMEM((NCH, IDX_CHUNK), jnp.int32),
            pltpu.VMEM((BW, D), jnp.float32),
            pltpu.VMEM((BW, D), jnp.float32),
            pltpu.VMEM((BW, L), jnp.float32),
            pltpu.VMEM((BW, L), jnp.float32),
            pltpu.VMEM((L,), jnp.float32),
            pltpu.VMEM((BW,), jnp.float32),
            pltpu.SemaphoreType.DMA,
        ],
    )
    def mf_kernel(u_hbm, i_hbm, U_hbm, V_hbm, bu_hbm, bi_hbm, gb_hbm,
                  out_hbm,
                  uix, iix, udiv, idiv, urows, vrows, bub, bib, gbv, outb,
                  sem):
        wid = lax.axis_index("s") * NC + lax.axis_index("c")
        pltpu.sync_copy(u_hbm.at[pl.ds(wid * NCH, NCH)], uix)
        pltpu.sync_copy(i_hbm.at[pl.ds(wid * NCH, NCH)], iix)
        pltpu.sync_copy(gb_hbm, gbv)

        copies = []
        for j in range(NCH):
            sl = pl.ds(j * IDX_CHUNK, IDX_CHUNK)
            copies.append(pltpu.async_copy(U_hbm.at[uix.at[j]], urows.at[sl], sem))
            copies.append(pltpu.async_copy(V_hbm.at[iix.at[j]], vrows.at[sl], sem))

        for j in range(NCH):
            for t in range(IDX_CHUNK // L):
                sl = pl.ds(t * L, L)
                udiv[j, sl] = lax.shift_right_logical(uix[j, sl], 4)
                idiv[j, sl] = lax.shift_right_logical(iix[j, sl], 4)
        for j in range(NCH):
            sl = pl.ds(j * IDX_CHUNK, IDX_CHUNK)
            copies.append(pltpu.async_copy(bu_hbm.at[udiv.at[j]], bub.at[sl], sem))
            copies.append(pltpu.async_copy(bi_hbm.at[idiv.at[j]], bib.at[sl], sem))
        for c in copies:
            c.wait()

        gb = gbv[...]
        iota = lax.iota(jnp.int32, L)
        GPC = IDX_CHUNK // L

        def group(g, carry):
            rows = g * L + iota
            acc = gb
            for d in range(D):
                cols = jnp.full((L,), d, jnp.int32)
                acc = acc + (plsc.load_gather(urows, [rows, cols])
                             * plsc.load_gather(vrows, [rows, cols]))
            j = jnp.full((L,), g // GPC, jnp.int32)
            off = (g % GPC) * L + iota
            ucols = plsc.load_gather(uix, [j, off]) & 15
            icols = plsc.load_gather(iix, [j, off]) & 15
            acc = (acc + plsc.load_gather(bub, [rows, ucols])
                   + plsc.load_gather(bib, [rows, icols]))
            outb[pl.ds(g * L, L)] = 1.0 + 4.0 / (1.0 + jnp.exp(-acc))
            return carry

        lax.fori_loop(0, BW // L, group, 0)
        pltpu.sync_copy(outb, out_hbm.at[pl.ds(wid * BW, BW)])

    return mf_kernel


def kernel(u, i, U, V, bu, bi, global_b):
    B = u.shape[0]
    f = _build(B)
    u2 = u.astype(jnp.int32).reshape(B // IDX_CHUNK, IDX_CHUNK)
    i2 = i.astype(jnp.int32).reshape(B // IDX_CHUNK, IDX_CHUNK)
    bu16 = bu.reshape(bu.shape[0] // L, L)
    bi16 = bi.reshape(bi.shape[0] // L, L)
    gb = jnp.broadcast_to(global_b.astype(jnp.float32), (L,))
    return f(u2, i2, U, V, bu16, bi16, gb)
